# baseline (speedup 1.0000x reference)
import functools
import jax
import jax.numpy as jnp
from jax import lax
from jax.experimental import pallas as pl
from jax.experimental.pallas import tpu as pltpu
from jax.experimental.pallas import tpu_sc as plsc

VOCAB = 50257
N_POS = 1024
D = 768
B = 64
S = 1024

NC = 2
NS = 16
NW = NC * NS
LANES = 16
D_SLICES = D // LANES

QB = 4
BL = B // QB
ST = 128
KC = 4
SC_W = ST // KC


def _body(ids_hbm, wte_hbm, wpe_hbm, out_hbm, idx_v, wpe_v, rows_v, sem):
    cid = lax.axis_index("c")
    sid = lax.axis_index("s")
    wid = sid * NC + cid
    t = wid // QB
    q = wid % QB

    pltpu.sync_copy(ids_hbm.at[pl.ds(q * BL, BL), pl.ds(t * ST, ST)], idx_v)

    def per_chunk(k, _):
        s0 = t * ST + k * SC_W
        pltpu.sync_copy(wpe_hbm.at[pl.ds(s0, SC_W), :], wpe_v)

        def per_batch(bb, _):
            idx = idx_v.at[bb, pl.ds(k * SC_W, SC_W)]
            pltpu.async_copy(wte_hbm.at[idx], rows_v, sem).wait()

            def per_row(r, _):
                for c in range(D_SLICES):
                    sl = pl.ds(c * LANES, LANES)
                    plsc.addupdate(rows_v.at[r, sl], wpe_v[r, sl])
                return _

            lax.fori_loop(0, SC_W, per_row, None)

            pltpu.sync_copy(rows_v, out_hbm.at[q * BL + bb, pl.ds(s0, SC_W), :])
            return _

        lax.fori_loop(0, BL, per_batch, None)
        return _

    lax.fori_loop(0, KC, per_chunk, None)


@jax.jit
def _embed(input_ids, wte, wpe):
    mesh = plsc.VectorSubcoreMesh(core_axis_name="c", subcore_axis_name="s")
    return pl.kernel(
        _body,
        out_type=jax.ShapeDtypeStruct((B, S, D), jnp.float32),
        mesh=mesh,
        scratch_types=[
            pltpu.VMEM((BL, ST), jnp.int32),
            pltpu.VMEM((SC_W, D), jnp.float32),
            pltpu.VMEM((SC_W, D), jnp.float32),
            pltpu.SemaphoreType.DMA,
        ],
    )(input_ids, wte, wpe)


def kernel(input_ids, attention_mask, wte, wpe):
    del attention_mask
    return _embed(input_ids, wte, wpe)
